# baseline (speedup 1.0000x reference)
import functools

import jax
import jax.numpy as jnp
from jax import lax
from jax.experimental import pallas as pl
from jax.experimental.pallas import tpu as pltpu
from jax.experimental.pallas import tpu_sc as plsc

B, L, K, H, S, V, TG = 8, 2048, 512, 128, 8, 100000, 64
NT = B * L
NPAIR = B * S
_LANES = 16

_NC = 2
_NS = 16
_NW = _NC * _NS


_TOK_PER_W = NT // _NW
_GCH = 128
_NGCH = _TOK_PER_W // _GCH


def _sc_gather_body(embed_hbm, tg_hbm, hist_hbm, post_hbm, auth_hbm, act_hbm,
                    gap_hbm, x0, x1, x2, x3, x4, idx_v, rows_v, sem):
    wid = lax.axis_index("s") * _NC + lax.axis_index("c")
    base = wid * _TOK_PER_W
    srcs = ((hist_hbm, embed_hbm, x0), (post_hbm, embed_hbm, x1),
            (auth_hbm, embed_hbm, x2), (act_hbm, embed_hbm, x3),
            (gap_hbm, tg_hbm, x4))
    for idx_hbm, table_hbm, out_hbm in srcs:
        for c in range(_NGCH):
            off = base + c * _GCH
            pltpu.sync_copy(idx_hbm.at[pl.ds(off, _GCH)], idx_v)
            pltpu.async_copy(table_hbm.at[idx_v], rows_v, sem).wait()
            pltpu.sync_copy(rows_v, out_hbm.at[pl.ds(off, _GCH)])


def _sc_gather(embed_table, tg_table, hist, post, auth, act, gap):
    mesh = plsc.VectorSubcoreMesh(core_axis_name="c", subcore_axis_name="s")
    xt = jax.ShapeDtypeStruct((NT, H), jnp.float32)
    fn = functools.partial(
        pl.kernel, mesh=mesh,
        out_type=[xt, xt, xt, xt, xt],
        compiler_params=pltpu.CompilerParams(needs_layout_passes=False),
        scratch_types=[
            pltpu.VMEM((_GCH,), jnp.int32),
            pltpu.VMEM((_GCH, H), jnp.float32),
            pltpu.SemaphoreType.DMA,
        ],
    )(_sc_gather_body)
    return fn(embed_table, tg_table, hist, post, auth, act, gap)


_BT = 1024


def _tc_mlp_body(x0, x1, x2, x3, x4, gamma, beta, w1, b1, w2, b2, out):
    x = jnp.concatenate(
        [x0[...], x1[...], x2[...], x3[...], x4[...]], axis=1)
    mu = jnp.mean(x, axis=-1, keepdims=True)
    var = jnp.mean((x - mu) ** 2, axis=-1, keepdims=True)
    xn = (x - mu) * lax.rsqrt(var + 1e-5) * gamma[...] + beta[...]
    h1 = jnp.dot(xn, w1[...], preferred_element_type=jnp.float32,
                 precision=lax.Precision.HIGHEST) + b1[...]
    h1 = h1 * jax.nn.sigmoid(h1)
    ev = jnp.dot(h1, w2[...], preferred_element_type=jnp.float32,
                 precision=lax.Precision.HIGHEST) + b2[...]
    out[...] = ev


def _tc_mlp(xs, ln_gamma, ln_beta, W1, b1, W2, b2):
    D5 = 5 * H
    D4 = 4 * H
    grid = (NT // _BT,)
    xspec = pl.BlockSpec((_BT, H), lambda i: (i, 0))

    def full(shape):
        return pl.BlockSpec(shape, lambda i: tuple(0 for _ in shape))

    return pl.pallas_call(
        _tc_mlp_body,
        grid=grid,
        in_specs=[xspec] * 5 + [full((1, D5)), full((1, D5)),
                                full((D5, D4)), full((1, D4)),
                                full((D4, H)), full((1, H))],
        out_specs=pl.BlockSpec((_BT, H), lambda i: (i, 0)),
        out_shape=jax.ShapeDtypeStruct((NT, H), jnp.float32),
    )(*xs, ln_gamma.reshape(1, D5), ln_beta.reshape(1, D5),
      W1, b1.reshape(1, D4), W2, b2.reshape(1, H))


_PCH = 128
_NPCH = K // _PCH
_NVL = L // _LANES
_PAIRS_PER_W = NPAIR // _NW


def _sc_pack_body(gid_hbm, msk_hbm, event_hbm, emp_hbm,
                  states_hbm, mout_hbm,
                  ids_v, msk_v, idxl_v, chunk_v, emp_v, mko_v, sem):
    wid = lax.axis_index("s") * _NC + lax.axis_index("c")
    iota = lax.iota(jnp.int32, _LANES)
    zero16 = jnp.zeros((_LANES,), jnp.int32)

    for p in range(_PAIRS_PER_W):
        pair = wid + p * _NW
        b = pair // S
        s = pair % S
        pltpu.sync_copy(gid_hbm.at[b], ids_v)
        pltpu.sync_copy(msk_hbm.at[b], msk_v)
        pltpu.sync_copy(emp_hbm.at[s], emp_v)

        sval = s + 1

        def count_body(j, c):
            ids16 = ids_v[pl.ds(j * _LANES, _LANES)]
            mk16 = msk_v[pl.ds(j * _LANES, _LANES)]
            m = (ids16 == sval) & (mk16 != 0)
            return c + jnp.sum(jnp.where(m, 1, 0))

        count = lax.fori_loop(0, _NVL, count_body, jnp.int32(0))
        start = jnp.maximum(count - K, 0)
        n = count - start

        def zidx_body(j, _):
            idxl_v[pl.ds(j * _LANES, _LANES)] = zero16
            return 0

        lax.fori_loop(0, K // _LANES, zidx_body, 0)

        def rank_body(j, c):
            ids16 = ids_v[pl.ds(j * _LANES, _LANES)]
            mk16 = msk_v[pl.ds(j * _LANES, _LANES)]
            m = (ids16 == sval) & (mk16 != 0)
            mi = jnp.where(m, 1, 0)
            rank = plsc.cumsum(mi) + c - 1
            slot = rank - start
            wm = m & (slot >= 0)
            slot_c = jnp.maximum(slot, 0)
            gidx = b * L + j * _LANES + iota
            plsc.store_scatter(idxl_v, [slot_c], gidx, mask=wm)
            return c + jnp.sum(mi)

        lax.fori_loop(0, _NVL, rank_body, jnp.int32(0))

        copies = []
        for c4 in range(_NPCH):
            k0 = c4 * _PCH
            copies.append(
                pltpu.async_copy(event_hbm.at[idxl_v.at[pl.ds(k0, _PCH)]],
                                 chunk_v.at[c4], sem))
        for c4 in range(_NPCH):
            copies[c4].wait()
            if c4 == 0:
                @pl.when(n == 0)
                def _():
                    for j2 in range(H // _LANES):
                        sl = pl.ds(j2 * _LANES, _LANES)
                        chunk_v[0, 0, sl] = emp_v[sl]

            pltpu.sync_copy(chunk_v.at[c4],
                            states_hbm.at[pl.ds(pair * K + c4 * _PCH, _PCH)])

        def mk_body(j, _):
            k16 = j * _LANES + iota
            mv = (k16 < n) | ((k16 == 0) & (n == 0))
            mko_v[pl.ds(j * _LANES, _LANES)] = jnp.where(mv, 1, 0)
            return 0

        lax.fori_loop(0, K // _LANES, mk_body, 0)
        pltpu.sync_copy(mko_v, mout_hbm.at[pair])


def _sc_pack(gid, maskI, event, empty_tokens):
    mesh = plsc.VectorSubcoreMesh(core_axis_name="c", subcore_axis_name="s")
    fn = functools.partial(
        pl.kernel, mesh=mesh,
        out_type=[jax.ShapeDtypeStruct((NPAIR * K, H), jnp.float32),
                  jax.ShapeDtypeStruct((NPAIR, K), jnp.int32)],
        compiler_params=pltpu.CompilerParams(needs_layout_passes=False),
        scratch_types=[
            pltpu.VMEM((L,), jnp.int32),
            pltpu.VMEM((L,), jnp.int32),
            pltpu.VMEM((K,), jnp.int32),
            pltpu.VMEM((_NPCH, _PCH, H), jnp.float32),
            pltpu.VMEM((H,), jnp.float32),
            pltpu.VMEM((K,), jnp.int32),
            pltpu.SemaphoreType.DMA,
        ],
    )(_sc_pack_body)
    return fn(gid, maskI, event, empty_tokens)


def _tc_final_body(raw, maskf, pos, sid, out):
    x = raw[0]
    m = maskf[0]
    out[0] = (x + pos[...][None, :, :] + sid[...][:, None, :]) * m


def _tc_final(raw, maskf, pos_table, sid_rows):
    return pl.pallas_call(
        _tc_final_body,
        grid=(B,),
        in_specs=[
            pl.BlockSpec((1, S, K, H), lambda i: (i, 0, 0, 0)),
            pl.BlockSpec((1, S, K, 1), lambda i: (i, 0, 0, 0)),
            pl.BlockSpec((K, H), lambda i: (0, 0)),
            pl.BlockSpec((S, H), lambda i: (0, 0)),
        ],
        out_specs=pl.BlockSpec((1, S, K, H), lambda i: (i, 0, 0, 0)),
        out_shape=jax.ShapeDtypeStruct((B, S, K, H), jnp.float32),
    )(raw, maskf, pos_table, sid_rows)


def kernel(history_tokens, history_post_tokens, history_author_tokens,
           history_action_tokens, history_time_gap, history_group_ids,
           history_mask, embed_table, time_gap_table, seq_id_table, pos_table,
           ln_gamma, ln_beta, W1, b1, W2, b2, empty_tokens):
    hist = history_tokens.reshape(NT).astype(jnp.int32)
    post = history_post_tokens.reshape(NT).astype(jnp.int32)
    auth = history_author_tokens.reshape(NT).astype(jnp.int32)
    act = history_action_tokens.reshape(NT).astype(jnp.int32)
    gap = history_time_gap.reshape(NT).astype(jnp.int32)
    gid = history_group_ids.astype(jnp.int32)
    maskI = history_mask.astype(jnp.int32)

    xs = _sc_gather(embed_table, time_gap_table, hist, post, auth, act, gap)
    event = _tc_mlp(xs, ln_gamma, ln_beta, W1, b1, W2, b2)
    sid_rows = seq_id_table[1:S + 1]
    raw, mout = _sc_pack(gid, maskI, event, empty_tokens)
    maskf = mout.astype(jnp.float32).reshape(B, S, K, 1)
    states = _tc_final(raw.reshape(B, S, K, H), maskf, pos_table, sid_rows)
    seq_mask = (mout != 0).reshape(B, S, K)
    return states, seq_mask


# speedup vs baseline: 4.3108x; 4.3108x over previous
import functools

import jax
import jax.numpy as jnp
from jax import lax
from jax.experimental import pallas as pl
from jax.experimental.pallas import tpu as pltpu
from jax.experimental.pallas import tpu_sc as plsc

B, L, K, H, S, V, TG = 8, 2048, 512, 128, 8, 100000, 64
NT = B * L
NPAIR = B * S
_LANES = 16

_NC = 2
_NS = 16
_NW = _NC * _NS


_TOK_PER_W = NT // _NW
_GCH = 128
_NGCH = _TOK_PER_W // _GCH


def _sc_gather_body(embed_hbm, tg_hbm, hist_hbm, post_hbm, auth_hbm, act_hbm,
                    gap_hbm, x0, x1, x2, x3, x4, idx_v, rows_v, sem):
    wid = lax.axis_index("s") * _NC + lax.axis_index("c")
    base = wid * _TOK_PER_W
    srcs = ((hist_hbm, embed_hbm, x0), (post_hbm, embed_hbm, x1),
            (auth_hbm, embed_hbm, x2), (act_hbm, embed_hbm, x3),
            (gap_hbm, tg_hbm, x4))
    for idx_hbm, table_hbm, out_hbm in srcs:
        for c in range(_NGCH):
            off = base + c * _GCH
            pltpu.sync_copy(idx_hbm.at[pl.ds(off, _GCH)], idx_v)
            pltpu.async_copy(table_hbm.at[idx_v], rows_v, sem).wait()
            pltpu.sync_copy(rows_v, out_hbm.at[pl.ds(off, _GCH)])


def _sc_gather(embed_table, tg_table, hist, post, auth, act, gap):
    mesh = plsc.VectorSubcoreMesh(core_axis_name="c", subcore_axis_name="s")
    xt = jax.ShapeDtypeStruct((NT, H), jnp.float32)
    fn = functools.partial(
        pl.kernel, mesh=mesh,
        out_type=[xt, xt, xt, xt, xt],
        compiler_params=pltpu.CompilerParams(needs_layout_passes=False),
        scratch_types=[
            pltpu.VMEM((_GCH,), jnp.int32),
            pltpu.VMEM((_GCH, H), jnp.float32),
            pltpu.SemaphoreType.DMA,
        ],
    )(_sc_gather_body)
    return fn(embed_table, tg_table, hist, post, auth, act, gap)


_BT = 1024


def _tc_mlp_body(x0, x1, x2, x3, x4, gamma, beta, w1, b1, w2, b2, out):
    x = jnp.concatenate(
        [x0[...], x1[...], x2[...], x3[...], x4[...]], axis=1)
    mu = jnp.mean(x, axis=-1, keepdims=True)
    var = jnp.mean((x - mu) ** 2, axis=-1, keepdims=True)
    xn = (x - mu) * lax.rsqrt(var + 1e-5) * gamma[...] + beta[...]
    h1 = jnp.dot(xn, w1[...], preferred_element_type=jnp.float32,
                 precision=lax.Precision.HIGHEST) + b1[...]
    h1 = h1 * jax.nn.sigmoid(h1)
    ev = jnp.dot(h1, w2[...], preferred_element_type=jnp.float32,
                 precision=lax.Precision.HIGHEST) + b2[...]
    out[...] = ev


def _tc_mlp(xs, ln_gamma, ln_beta, W1, b1, W2, b2):
    D5 = 5 * H
    D4 = 4 * H
    grid = (NT // _BT,)
    xspec = pl.BlockSpec((_BT, H), lambda i: (i, 0))

    def full(shape):
        return pl.BlockSpec(shape, lambda i: tuple(0 for _ in shape))

    return pl.pallas_call(
        _tc_mlp_body,
        grid=grid,
        in_specs=[xspec] * 5 + [full((1, D5)), full((1, D5)),
                                full((D5, D4)), full((1, D4)),
                                full((D4, H)), full((1, H))],
        out_specs=pl.BlockSpec((_BT, H), lambda i: (i, 0)),
        out_shape=jax.ShapeDtypeStruct((NT, H), jnp.float32),
    )(*xs, ln_gamma.reshape(1, D5), ln_beta.reshape(1, D5),
      W1, b1.reshape(1, D4), W2, b2.reshape(1, H))


_PCH = 128
_NPCH = K // _PCH
_NVL = L // _LANES
_PAIRS_PER_W = NPAIR // _NW


def _sc_pack_body(gid_hbm, msk_hbm, event_hbm, emp_hbm,
                  states_hbm, mout_hbm,
                  ids_v, msk_v, idxl_v, chunk_v, emp_v, mko_v, sem):
    wid = lax.axis_index("s") * _NC + lax.axis_index("c")
    iota = lax.iota(jnp.int32, _LANES)
    zero16 = jnp.zeros((_LANES,), jnp.int32)

    for p in range(_PAIRS_PER_W):
        pair = wid + p * _NW
        b = pair // S
        s = pair % S
        pltpu.sync_copy(gid_hbm.at[b], ids_v)
        pltpu.sync_copy(msk_hbm.at[b], msk_v)
        pltpu.sync_copy(emp_hbm.at[s], emp_v)

        sval = s + 1

        def count_body(j, c):
            ids16 = ids_v[pl.ds(j * _LANES, _LANES)]
            mk16 = msk_v[pl.ds(j * _LANES, _LANES)]
            m = (ids16 == sval) & (mk16 != 0)
            return c + jnp.sum(jnp.where(m, 1, 0))

        count = lax.fori_loop(0, _NVL, count_body, jnp.int32(0))
        start = jnp.maximum(count - K, 0)
        n = count - start

        def zidx_body(j, _):
            idxl_v[pl.ds(j * _LANES, _LANES)] = j * _LANES + iota
            return 0

        lax.fori_loop(0, K // _LANES, zidx_body, 0)

        def rank_body(j, c):
            ids16 = ids_v[pl.ds(j * _LANES, _LANES)]
            mk16 = msk_v[pl.ds(j * _LANES, _LANES)]
            m = (ids16 == sval) & (mk16 != 0)
            mi = jnp.where(m, 1, 0)
            rank = plsc.cumsum(mi) + c - 1
            slot = rank - start
            wm = m & (slot >= 0)
            slot_c = jnp.maximum(slot, 0)
            gidx = b * L + j * _LANES + iota
            plsc.store_scatter(idxl_v, [slot_c], gidx, mask=wm)
            return c + jnp.sum(mi)

        lax.fori_loop(0, _NVL, rank_body, jnp.int32(0))

        for c4 in range(_NPCH):
            k0 = c4 * _PCH

            @pl.when(k0 < n)
            def _(c4=c4, k0=k0):
                pltpu.async_copy(event_hbm.at[idxl_v.at[pl.ds(k0, _PCH)]],
                                 chunk_v.at[c4], sem)

        for c4 in range(_NPCH):
            k0 = c4 * _PCH

            @pl.when(k0 < n)
            def _(c4=c4, k0=k0):
                pltpu.make_async_copy(
                    event_hbm.at[idxl_v.at[pl.ds(k0, _PCH)]],
                    chunk_v.at[c4], sem).wait()

            if c4 == 0:
                @pl.when(n == 0)
                def _():
                    for j2 in range(H // _LANES):
                        sl = pl.ds(j2 * _LANES, _LANES)
                        chunk_v[0, 0, sl] = emp_v[sl]

            if c4 == 0:
                pltpu.sync_copy(chunk_v.at[0],
                                states_hbm.at[pl.ds(pair * K, _PCH)])
            else:
                @pl.when(k0 < n)
                def _(c4=c4, k0=k0):
                    pltpu.sync_copy(chunk_v.at[c4],
                                    states_hbm.at[pl.ds(pair * K + k0, _PCH)])

        def mk_body(j, _):
            k16 = j * _LANES + iota
            mv = (k16 < n) | ((k16 == 0) & (n == 0))
            mko_v[pl.ds(j * _LANES, _LANES)] = jnp.where(mv, 1, 0)
            return 0

        lax.fori_loop(0, K // _LANES, mk_body, 0)
        pltpu.sync_copy(mko_v, mout_hbm.at[pair])


def _sc_pack(gid, maskI, event, empty_tokens):
    mesh = plsc.VectorSubcoreMesh(core_axis_name="c", subcore_axis_name="s")
    fn = functools.partial(
        pl.kernel, mesh=mesh,
        out_type=[jax.ShapeDtypeStruct((NPAIR * K, H), jnp.float32),
                  jax.ShapeDtypeStruct((NPAIR, K), jnp.int32)],
        compiler_params=pltpu.CompilerParams(needs_layout_passes=False),
        scratch_types=[
            pltpu.VMEM((L,), jnp.int32),
            pltpu.VMEM((L,), jnp.int32),
            pltpu.VMEM((K,), jnp.int32),
            pltpu.VMEM((_NPCH, _PCH, H), jnp.float32),
            pltpu.VMEM((H,), jnp.float32),
            pltpu.VMEM((K,), jnp.int32),
            pltpu.SemaphoreType.DMA,
        ],
    )(_sc_pack_body)
    return fn(gid, maskI, event, empty_tokens)


def _tc_final_body(raw, maskf, pos, sid, out):
    x = raw[0]
    m = maskf[0]
    val = x + pos[...][None, :, :] + sid[...][:, None, :]
    out[0] = jnp.where(m > 0.0, val, 0.0)


def _tc_final(raw, maskf, pos_table, sid_rows):
    return pl.pallas_call(
        _tc_final_body,
        grid=(B,),
        in_specs=[
            pl.BlockSpec((1, S, K, H), lambda i: (i, 0, 0, 0)),
            pl.BlockSpec((1, S, K, 1), lambda i: (i, 0, 0, 0)),
            pl.BlockSpec((K, H), lambda i: (0, 0)),
            pl.BlockSpec((S, H), lambda i: (0, 0)),
        ],
        out_specs=pl.BlockSpec((1, S, K, H), lambda i: (i, 0, 0, 0)),
        out_shape=jax.ShapeDtypeStruct((B, S, K, H), jnp.float32),
    )(raw, maskf, pos_table, sid_rows)


def kernel(history_tokens, history_post_tokens, history_author_tokens,
           history_action_tokens, history_time_gap, history_group_ids,
           history_mask, embed_table, time_gap_table, seq_id_table, pos_table,
           ln_gamma, ln_beta, W1, b1, W2, b2, empty_tokens):
    hist = history_tokens.reshape(NT).astype(jnp.int32)
    post = history_post_tokens.reshape(NT).astype(jnp.int32)
    auth = history_author_tokens.reshape(NT).astype(jnp.int32)
    act = history_action_tokens.reshape(NT).astype(jnp.int32)
    gap = history_time_gap.reshape(NT).astype(jnp.int32)
    gid = history_group_ids.astype(jnp.int32)
    maskI = history_mask.astype(jnp.int32)

    xs = _sc_gather(embed_table, time_gap_table, hist, post, auth, act, gap)
    event = _tc_mlp(xs, ln_gamma, ln_beta, W1, b1, W2, b2)
    sid_rows = seq_id_table[1:S + 1]
    raw, mout = _sc_pack(gid, maskI, event, empty_tokens)
    maskf = mout.astype(jnp.float32).reshape(B, S, K, 1)
    states = _tc_final(raw.reshape(B, S, K, H), maskf, pos_table, sid_rows)
    seq_mask = (mout != 0).reshape(B, S, K)
    return states, seq_mask


# speedup vs baseline: 7.0876x; 1.6441x over previous
import functools

import jax
import jax.numpy as jnp
from jax import lax
from jax.experimental import pallas as pl
from jax.experimental.pallas import tpu as pltpu
from jax.experimental.pallas import tpu_sc as plsc

B, L, K, H, S, V, TG = 8, 2048, 512, 128, 8, 100000, 64
NT = B * L
NPAIR = B * S
_LANES = 16

_NC = 2
_NS = 16
_NW = _NC * _NS


_TOK_PER_W = NT // _NW
_GCH = 128
_NGCH = _TOK_PER_W // _GCH


def _sc_gather_body(embed_hbm, tg_hbm, hist_hbm, post_hbm, auth_hbm, act_hbm,
                    gap_hbm, x0, x1, x2, x3, x4, idx_v, rows_v, sem):
    wid = lax.axis_index("s") * _NC + lax.axis_index("c")
    base = wid * _TOK_PER_W
    srcs = ((hist_hbm, embed_hbm, x0), (post_hbm, embed_hbm, x1),
            (auth_hbm, embed_hbm, x2), (act_hbm, embed_hbm, x3),
            (gap_hbm, tg_hbm, x4))
    for idx_hbm, table_hbm, out_hbm in srcs:
        for c in range(_NGCH):
            off = base + c * _GCH
            pltpu.sync_copy(idx_hbm.at[pl.ds(off, _GCH)], idx_v)
            pltpu.async_copy(table_hbm.at[idx_v], rows_v, sem).wait()
            pltpu.sync_copy(rows_v, out_hbm.at[pl.ds(off, _GCH)])


def _sc_gather(embed_table, tg_table, hist, post, auth, act, gap):
    mesh = plsc.VectorSubcoreMesh(core_axis_name="c", subcore_axis_name="s")
    xt = jax.ShapeDtypeStruct((NT, H), jnp.float32)
    fn = functools.partial(
        pl.kernel, mesh=mesh,
        out_type=[xt, xt, xt, xt, xt],
        compiler_params=pltpu.CompilerParams(needs_layout_passes=False),
        scratch_types=[
            pltpu.VMEM((_GCH,), jnp.int32),
            pltpu.VMEM((_GCH, H), jnp.float32),
            pltpu.SemaphoreType.DMA,
        ],
    )(_sc_gather_body)
    return fn(embed_table, tg_table, hist, post, auth, act, gap)


_BT = 1024


def _tc_mlp_body(x0, x1, x2, x3, x4, gamma, beta, w1, b1, w2, b2, out):
    x = jnp.concatenate(
        [x0[...], x1[...], x2[...], x3[...], x4[...]], axis=1)
    mu = jnp.mean(x, axis=-1, keepdims=True)
    var = jnp.mean((x - mu) ** 2, axis=-1, keepdims=True)
    xn = (x - mu) * lax.rsqrt(var + 1e-5) * gamma[...] + beta[...]
    h1 = jnp.dot(xn, w1[...], preferred_element_type=jnp.float32) + b1[...]
    h1 = h1 * jax.nn.sigmoid(h1)
    ev = jnp.dot(h1, w2[...], preferred_element_type=jnp.float32) + b2[...]
    out[...] = ev


def _tc_mlp(xs, ln_gamma, ln_beta, W1, b1, W2, b2):
    D5 = 5 * H
    D4 = 4 * H
    grid = (NT // _BT,)
    xspec = pl.BlockSpec((_BT, H), lambda i: (i, 0))

    def full(shape):
        return pl.BlockSpec(shape, lambda i: tuple(0 for _ in shape))

    return pl.pallas_call(
        _tc_mlp_body,
        grid=grid,
        in_specs=[xspec] * 5 + [full((1, D5)), full((1, D5)),
                                full((D5, D4)), full((1, D4)),
                                full((D4, H)), full((1, H))],
        out_specs=pl.BlockSpec((_BT, H), lambda i: (i, 0)),
        out_shape=jax.ShapeDtypeStruct((NT, H), jnp.float32),
    )(*xs, ln_gamma.reshape(1, D5), ln_beta.reshape(1, D5),
      W1, b1.reshape(1, D4), W2, b2.reshape(1, H))


_PCH = 128
_NPCH = K // _PCH
_NVL = L // _LANES
_PAIRS_PER_W = NPAIR // _NW


def _sc_pack_body(gid_hbm, msk_hbm, event_hbm, emp_hbm,
                  states_hbm, mout_hbm,
                  ids_v, msk_v, idxl_v, chunk_v, emp_v, mko_v, sem):
    wid = lax.axis_index("s") * _NC + lax.axis_index("c")
    iota = lax.iota(jnp.int32, _LANES)
    zero16 = jnp.zeros((_LANES,), jnp.int32)

    for p in range(_PAIRS_PER_W):
        pair = wid + p * _NW
        b = pair // S
        s = pair % S
        pltpu.sync_copy(gid_hbm.at[b], ids_v)
        pltpu.sync_copy(msk_hbm.at[b], msk_v)
        pltpu.sync_copy(emp_hbm.at[s], emp_v)

        sval = s + 1

        def count_body(j, c):
            ids16 = ids_v[pl.ds(j * _LANES, _LANES)]
            mk16 = msk_v[pl.ds(j * _LANES, _LANES)]
            m = (ids16 == sval) & (mk16 != 0)
            return c + jnp.sum(jnp.where(m, 1, 0))

        count = lax.fori_loop(0, _NVL, count_body, jnp.int32(0))
        start = jnp.maximum(count - K, 0)
        n = count - start

        def zidx_body(j, _):
            idxl_v[pl.ds(j * _LANES, _LANES)] = j * _LANES + iota
            return 0

        lax.fori_loop(0, K // _LANES, zidx_body, 0)

        def rank_body(j, c):
            ids16 = ids_v[pl.ds(j * _LANES, _LANES)]
            mk16 = msk_v[pl.ds(j * _LANES, _LANES)]
            m = (ids16 == sval) & (mk16 != 0)
            mi = jnp.where(m, 1, 0)
            rank = plsc.cumsum(mi) + c - 1
            slot = rank - start
            wm = m & (slot >= 0)
            slot_c = jnp.maximum(slot, 0)
            gidx = b * L + j * _LANES + iota
            plsc.store_scatter(idxl_v, [slot_c], gidx, mask=wm)
            return c + jnp.sum(mi)

        lax.fori_loop(0, _NVL, rank_body, jnp.int32(0))

        for c4 in range(_NPCH):
            k0 = c4 * _PCH

            @pl.when(k0 < n)
            def _(c4=c4, k0=k0):
                pltpu.async_copy(event_hbm.at[idxl_v.at[pl.ds(k0, _PCH)]],
                                 chunk_v.at[c4], sem)

        for c4 in range(_NPCH):
            k0 = c4 * _PCH

            @pl.when(k0 < n)
            def _(c4=c4, k0=k0):
                pltpu.make_async_copy(
                    event_hbm.at[idxl_v.at[pl.ds(k0, _PCH)]],
                    chunk_v.at[c4], sem).wait()

            if c4 == 0:
                @pl.when(n == 0)
                def _():
                    for j2 in range(H // _LANES):
                        sl = pl.ds(j2 * _LANES, _LANES)
                        chunk_v[0, 0, sl] = emp_v[sl]

            if c4 == 0:
                pltpu.sync_copy(chunk_v.at[0],
                                states_hbm.at[pl.ds(pair * K, _PCH)])
            else:
                @pl.when(k0 < n)
                def _(c4=c4, k0=k0):
                    pltpu.sync_copy(chunk_v.at[c4],
                                    states_hbm.at[pl.ds(pair * K + k0, _PCH)])

        def mk_body(j, _):
            k16 = j * _LANES + iota
            mv = (k16 < n) | ((k16 == 0) & (n == 0))
            mko_v[pl.ds(j * _LANES, _LANES)] = jnp.where(mv, 1, 0)
            return 0

        lax.fori_loop(0, K // _LANES, mk_body, 0)
        pltpu.sync_copy(mko_v, mout_hbm.at[pair])


def _sc_pack(gid, maskI, event, empty_tokens):
    mesh = plsc.VectorSubcoreMesh(core_axis_name="c", subcore_axis_name="s")
    fn = functools.partial(
        pl.kernel, mesh=mesh,
        out_type=[jax.ShapeDtypeStruct((NPAIR * K, H), jnp.float32),
                  jax.ShapeDtypeStruct((NPAIR, K), jnp.int32)],
        compiler_params=pltpu.CompilerParams(needs_layout_passes=False),
        scratch_types=[
            pltpu.VMEM((L,), jnp.int32),
            pltpu.VMEM((L,), jnp.int32),
            pltpu.VMEM((K,), jnp.int32),
            pltpu.VMEM((_NPCH, _PCH, H), jnp.float32),
            pltpu.VMEM((H,), jnp.float32),
            pltpu.VMEM((K,), jnp.int32),
            pltpu.SemaphoreType.DMA,
        ],
    )(_sc_pack_body)
    return fn(gid, maskI, event, empty_tokens)


def _tc_final_body(raw, maskf, pos, sid, out):
    x = raw[0]
    m = maskf[0]
    val = x + pos[...][None, :, :] + sid[...][:, None, :]
    out[0] = jnp.where(m > 0.0, val, 0.0)


def _tc_final(raw, maskf, pos_table, sid_rows):
    return pl.pallas_call(
        _tc_final_body,
        grid=(B,),
        in_specs=[
            pl.BlockSpec((1, S, K, H), lambda i: (i, 0, 0, 0)),
            pl.BlockSpec((1, S, K, 1), lambda i: (i, 0, 0, 0)),
            pl.BlockSpec((K, H), lambda i: (0, 0)),
            pl.BlockSpec((S, H), lambda i: (0, 0)),
        ],
        out_specs=pl.BlockSpec((1, S, K, H), lambda i: (i, 0, 0, 0)),
        out_shape=jax.ShapeDtypeStruct((B, S, K, H), jnp.float32),
    )(raw, maskf, pos_table, sid_rows)


def kernel(history_tokens, history_post_tokens, history_author_tokens,
           history_action_tokens, history_time_gap, history_group_ids,
           history_mask, embed_table, time_gap_table, seq_id_table, pos_table,
           ln_gamma, ln_beta, W1, b1, W2, b2, empty_tokens):
    hist = history_tokens.reshape(NT).astype(jnp.int32)
    post = history_post_tokens.reshape(NT).astype(jnp.int32)
    auth = history_author_tokens.reshape(NT).astype(jnp.int32)
    act = history_action_tokens.reshape(NT).astype(jnp.int32)
    gap = history_time_gap.reshape(NT).astype(jnp.int32)
    gid = history_group_ids.astype(jnp.int32)
    maskI = history_mask.astype(jnp.int32)

    xs = _sc_gather(embed_table, time_gap_table, hist, post, auth, act, gap)
    event = _tc_mlp(xs, ln_gamma, ln_beta, W1, b1, W2, b2)
    sid_rows = seq_id_table[1:S + 1]
    raw, mout = _sc_pack(gid, maskI, event, empty_tokens)
    maskf = mout.astype(jnp.float32).reshape(B, S, K, 1)
    states = _tc_final(raw.reshape(B, S, K, H), maskf, pos_table, sid_rows)
    seq_mask = (mout != 0).reshape(B, S, K)
    return states, seq_mask


# speedup vs baseline: 7.0933x; 1.0008x over previous
import functools

import jax
import jax.numpy as jnp
from jax import lax
from jax.experimental import pallas as pl
from jax.experimental.pallas import tpu as pltpu
from jax.experimental.pallas import tpu_sc as plsc

B, L, K, H, S, V, TG = 8, 2048, 512, 128, 8, 100000, 64
NT = B * L
NPAIR = B * S
_LANES = 16

_NC = 2
_NS = 16
_NW = _NC * _NS


_GCH = 128


def _make_sc_gather(nt):
    tok_per_w = nt // _NW
    ngch = tok_per_w // _GCH

    def body(embed_hbm, tg_hbm, hist_hbm, post_hbm, auth_hbm, act_hbm,
             gap_hbm, x0, x1, x2, x3, x4, idx_v, rows_v, sem):
        wid = lax.axis_index("s") * _NC + lax.axis_index("c")
        base = wid * tok_per_w
        srcs = ((hist_hbm, embed_hbm, x0), (post_hbm, embed_hbm, x1),
                (auth_hbm, embed_hbm, x2), (act_hbm, embed_hbm, x3),
                (gap_hbm, tg_hbm, x4))
        for idx_hbm, table_hbm, out_hbm in srcs:
            for c in range(ngch):
                off = base + c * _GCH
                pltpu.sync_copy(idx_hbm.at[pl.ds(off, _GCH)], idx_v)
                pltpu.async_copy(table_hbm.at[idx_v], rows_v, sem).wait()
                pltpu.sync_copy(rows_v, out_hbm.at[pl.ds(off, _GCH)])

    mesh = plsc.VectorSubcoreMesh(core_axis_name="c", subcore_axis_name="s")
    xt = jax.ShapeDtypeStruct((nt, H), jnp.float32)
    return functools.partial(
        pl.kernel, mesh=mesh,
        out_type=[xt, xt, xt, xt, xt],
        compiler_params=pltpu.CompilerParams(needs_layout_passes=False),
        scratch_types=[
            pltpu.VMEM((_GCH,), jnp.int32),
            pltpu.VMEM((_GCH, H), jnp.float32),
            pltpu.SemaphoreType.DMA,
        ],
    )(body)


_BT = 1024


def _tc_mlp_body(x0, x1, x2, x3, x4, gamma, beta, w1, b1, w2, b2, out):
    x = jnp.concatenate(
        [x0[...], x1[...], x2[...], x3[...], x4[...]], axis=1)
    mu = jnp.mean(x, axis=-1, keepdims=True)
    var = jnp.mean((x - mu) ** 2, axis=-1, keepdims=True)
    xn = (x - mu) * lax.rsqrt(var + 1e-5) * gamma[...] + beta[...]
    h1 = jnp.dot(xn, w1[...], preferred_element_type=jnp.float32) + b1[...]
    h1 = h1 * jax.nn.sigmoid(h1)
    ev = jnp.dot(h1, w2[...], preferred_element_type=jnp.float32) + b2[...]
    out[...] = ev


def _tc_mlp(xs, ln_gamma, ln_beta, W1, b1, W2, b2, nt=NT):
    D5 = 5 * H
    D4 = 4 * H
    grid = (nt // _BT,)
    xspec = pl.BlockSpec((_BT, H), lambda i: (i, 0))

    def full(shape):
        return pl.BlockSpec(shape, lambda i: tuple(0 for _ in shape))

    return pl.pallas_call(
        _tc_mlp_body,
        grid=grid,
        in_specs=[xspec] * 5 + [full((1, D5)), full((1, D5)),
                                full((D5, D4)), full((1, D4)),
                                full((D4, H)), full((1, H))],
        out_specs=pl.BlockSpec((_BT, H), lambda i: (i, 0)),
        out_shape=jax.ShapeDtypeStruct((nt, H), jnp.float32),
    )(*xs, ln_gamma.reshape(1, D5), ln_beta.reshape(1, D5),
      W1, b1.reshape(1, D4), W2, b2.reshape(1, H))


_PCH = 128
_NPCH = K // _PCH
_NVL = L // _LANES
_PAIRS_PER_W = NPAIR // _NW


def _sc_pack_body(gid_hbm, msk_hbm, event_hbm, emp_hbm,
                  states_hbm, mout_hbm,
                  ids_v, msk_v, idxl_v, chunk_v, emp_v, mko_v, sem):
    wid = lax.axis_index("s") * _NC + lax.axis_index("c")
    iota = lax.iota(jnp.int32, _LANES)
    zero16 = jnp.zeros((_LANES,), jnp.int32)

    for p in range(_PAIRS_PER_W):
        pair = wid + p * _NW
        b = pair // S
        s = pair % S
        pltpu.sync_copy(gid_hbm.at[b], ids_v)
        pltpu.sync_copy(msk_hbm.at[b], msk_v)
        pltpu.sync_copy(emp_hbm.at[s], emp_v)

        sval = s + 1

        def count_body(j, c):
            ids16 = ids_v[pl.ds(j * _LANES, _LANES)]
            mk16 = msk_v[pl.ds(j * _LANES, _LANES)]
            m = (ids16 == sval) & (mk16 != 0)
            return c + jnp.sum(jnp.where(m, 1, 0))

        count = lax.fori_loop(0, _NVL, count_body, jnp.int32(0))
        start = jnp.maximum(count - K, 0)
        n = count - start

        def zidx_body(j, _):
            idxl_v[pl.ds(j * _LANES, _LANES)] = j * _LANES + iota
            return 0

        lax.fori_loop(0, K // _LANES, zidx_body, 0)

        def rank_body(j, c):
            ids16 = ids_v[pl.ds(j * _LANES, _LANES)]
            mk16 = msk_v[pl.ds(j * _LANES, _LANES)]
            m = (ids16 == sval) & (mk16 != 0)
            mi = jnp.where(m, 1, 0)
            rank = plsc.cumsum(mi) + c - 1
            slot = rank - start
            wm = m & (slot >= 0)
            slot_c = jnp.maximum(slot, 0)
            gidx = b * L + j * _LANES + iota
            plsc.store_scatter(idxl_v, [slot_c], gidx, mask=wm)
            return c + jnp.sum(mi)

        lax.fori_loop(0, _NVL, rank_body, jnp.int32(0))

        for c4 in range(_NPCH):
            k0 = c4 * _PCH

            @pl.when(k0 < n)
            def _(c4=c4, k0=k0):
                pltpu.async_copy(event_hbm.at[idxl_v.at[pl.ds(k0, _PCH)]],
                                 chunk_v.at[c4], sem)

        for c4 in range(_NPCH):
            k0 = c4 * _PCH

            @pl.when(k0 < n)
            def _(c4=c4, k0=k0):
                pltpu.make_async_copy(
                    event_hbm.at[idxl_v.at[pl.ds(k0, _PCH)]],
                    chunk_v.at[c4], sem).wait()

            if c4 == 0:
                @pl.when(n == 0)
                def _():
                    for j2 in range(H // _LANES):
                        sl = pl.ds(j2 * _LANES, _LANES)
                        chunk_v[0, 0, sl] = emp_v[sl]

            if c4 == 0:
                pltpu.sync_copy(chunk_v.at[0],
                                states_hbm.at[pl.ds(pair * K, _PCH)])
            else:
                @pl.when(k0 < n)
                def _(c4=c4, k0=k0):
                    pltpu.sync_copy(chunk_v.at[c4],
                                    states_hbm.at[pl.ds(pair * K + k0, _PCH)])

        def mk_body(j, _):
            k16 = j * _LANES + iota
            mv = (k16 < n) | ((k16 == 0) & (n == 0))
            mko_v[pl.ds(j * _LANES, _LANES)] = jnp.where(mv, 1, 0)
            return 0

        lax.fori_loop(0, K // _LANES, mk_body, 0)
        pltpu.sync_copy(mko_v, mout_hbm.at[pair])


def _sc_pack(gid, maskI, event, empty_tokens):
    mesh = plsc.VectorSubcoreMesh(core_axis_name="c", subcore_axis_name="s")
    fn = functools.partial(
        pl.kernel, mesh=mesh,
        out_type=[jax.ShapeDtypeStruct((NPAIR * K, H), jnp.float32),
                  jax.ShapeDtypeStruct((NPAIR, K), jnp.int32)],
        compiler_params=pltpu.CompilerParams(needs_layout_passes=False),
        scratch_types=[
            pltpu.VMEM((L,), jnp.int32),
            pltpu.VMEM((L,), jnp.int32),
            pltpu.VMEM((K,), jnp.int32),
            pltpu.VMEM((_NPCH, _PCH, H), jnp.float32),
            pltpu.VMEM((H,), jnp.float32),
            pltpu.VMEM((K,), jnp.int32),
            pltpu.SemaphoreType.DMA,
        ],
    )(_sc_pack_body)
    return fn(gid, maskI, event, empty_tokens)


def _tc_final_body(raw, maskf, pos, sid, out):
    x = raw[0]
    m = maskf[0]
    val = x + pos[...][None, :, :] + sid[...][:, None, :]
    out[0] = jnp.where(m > 0.0, val, 0.0)


def _tc_final(raw, maskf, pos_table, sid_rows):
    return pl.pallas_call(
        _tc_final_body,
        grid=(B,),
        in_specs=[
            pl.BlockSpec((1, S, K, H), lambda i: (i, 0, 0, 0)),
            pl.BlockSpec((1, S, K, 1), lambda i: (i, 0, 0, 0)),
            pl.BlockSpec((K, H), lambda i: (0, 0)),
            pl.BlockSpec((S, H), lambda i: (0, 0)),
        ],
        out_specs=pl.BlockSpec((1, S, K, H), lambda i: (i, 0, 0, 0)),
        out_shape=jax.ShapeDtypeStruct((B, S, K, H), jnp.float32),
    )(raw, maskf, pos_table, sid_rows)


def kernel(history_tokens, history_post_tokens, history_author_tokens,
           history_action_tokens, history_time_gap, history_group_ids,
           history_mask, embed_table, time_gap_table, seq_id_table, pos_table,
           ln_gamma, ln_beta, W1, b1, W2, b2, empty_tokens):
    hist = history_tokens.reshape(NT).astype(jnp.int32)
    post = history_post_tokens.reshape(NT).astype(jnp.int32)
    auth = history_author_tokens.reshape(NT).astype(jnp.int32)
    act = history_action_tokens.reshape(NT).astype(jnp.int32)
    gap = history_time_gap.reshape(NT).astype(jnp.int32)
    gid = history_group_ids.astype(jnp.int32)
    maskI = history_mask.astype(jnp.int32)

    half = NT // 2
    gfn = _make_sc_gather(half)
    xs_a = gfn(embed_table, time_gap_table, hist[:half], post[:half],
               auth[:half], act[:half], gap[:half])
    xs_b = gfn(embed_table, time_gap_table, hist[half:], post[half:],
               auth[half:], act[half:], gap[half:])
    ev_a = _tc_mlp(xs_a, ln_gamma, ln_beta, W1, b1, W2, b2, nt=half)
    ev_b = _tc_mlp(xs_b, ln_gamma, ln_beta, W1, b1, W2, b2, nt=half)
    event = jnp.concatenate([ev_a, ev_b], axis=0)
    sid_rows = seq_id_table[1:S + 1]
    raw, mout = _sc_pack(gid, maskI, event, empty_tokens)
    maskf = mout.astype(jnp.float32).reshape(B, S, K, 1)
    states = _tc_final(raw.reshape(B, S, K, H), maskf, pos_table, sid_rows)
    seq_mask = (mout != 0).reshape(B, S, K)
    return states, seq_mask


# speedup vs baseline: 7.8554x; 1.1074x over previous
import functools

import jax
import jax.numpy as jnp
from jax import lax
from jax.experimental import pallas as pl
from jax.experimental.pallas import tpu as pltpu
from jax.experimental.pallas import tpu_sc as plsc

B, L, K, H, S, V, TG = 8, 2048, 512, 128, 8, 100000, 64
NT = B * L
NPAIR = B * S
_LANES = 16

_NC = 2
_NS = 16
_NW = _NC * _NS


_GCH = 128


def _make_sc_gather(nt):
    tok_per_w = nt // _NW
    ngch = tok_per_w // _GCH

    def body(embed_hbm, hist_hbm, post_hbm, auth_hbm, act_hbm,
             x0, x1, x2, x3, idx_v, rows_v, sem):
        wid = lax.axis_index("s") * _NC + lax.axis_index("c")
        base = wid * tok_per_w
        srcs = ((hist_hbm, x0), (post_hbm, x1), (auth_hbm, x2), (act_hbm, x3))
        for idx_hbm, out_hbm in srcs:
            for c in range(ngch):
                off = base + c * _GCH
                pltpu.sync_copy(idx_hbm.at[pl.ds(off, _GCH)], idx_v)
                pltpu.async_copy(embed_hbm.at[idx_v], rows_v, sem).wait()
                pltpu.sync_copy(rows_v, out_hbm.at[pl.ds(off, _GCH)])

    mesh = plsc.VectorSubcoreMesh(core_axis_name="c", subcore_axis_name="s")
    xt = jax.ShapeDtypeStruct((nt, H), jnp.float32)
    return functools.partial(
        pl.kernel, mesh=mesh,
        out_type=[xt, xt, xt, xt],
        compiler_params=pltpu.CompilerParams(needs_layout_passes=False),
        scratch_types=[
            pltpu.VMEM((_GCH,), jnp.int32),
            pltpu.VMEM((_GCH, H), jnp.float32),
            pltpu.SemaphoreType.DMA,
        ],
    )(body)


_BT = 1024


def _tc_mlp_body(x0, x1, x2, x3, gap, tgp, gamma, beta, w1, b1, w2, b2, out):
    oh = jnp.where(gap[...] == lax.broadcasted_iota(jnp.int32, (_BT, H), 1),
                   1.0, 0.0)
    x4 = jnp.dot(oh, tgp[...], preferred_element_type=jnp.float32,
                 precision=lax.Precision.HIGHEST)
    x = jnp.concatenate(
        [x0[...], x1[...], x2[...], x3[...], x4], axis=1)
    mu = jnp.mean(x, axis=-1, keepdims=True)
    var = jnp.mean((x - mu) ** 2, axis=-1, keepdims=True)
    xn = (x - mu) * lax.rsqrt(var + 1e-5) * gamma[...] + beta[...]
    h1 = jnp.dot(xn, w1[...], preferred_element_type=jnp.float32) + b1[...]
    h1 = h1 * jax.nn.sigmoid(h1)
    ev = jnp.dot(h1, w2[...], preferred_element_type=jnp.float32) + b2[...]
    out[...] = ev


def _tc_mlp(xs, gap2d, tgpad, ln_gamma, ln_beta, W1, b1, W2, b2, nt=NT):
    D5 = 5 * H
    D4 = 4 * H
    grid = (nt // _BT,)
    xspec = pl.BlockSpec((_BT, H), lambda i: (i, 0))

    def full(shape):
        return pl.BlockSpec(shape, lambda i: tuple(0 for _ in shape))

    return pl.pallas_call(
        _tc_mlp_body,
        grid=grid,
        in_specs=[xspec] * 4 + [pl.BlockSpec((_BT, 1), lambda i: (i, 0)),
                                full((H, H)), full((1, D5)), full((1, D5)),
                                full((D5, D4)), full((1, D4)),
                                full((D4, H)), full((1, H))],
        out_specs=pl.BlockSpec((_BT, H), lambda i: (i, 0)),
        out_shape=jax.ShapeDtypeStruct((nt, H), jnp.float32),
    )(*xs, gap2d, tgpad, ln_gamma.reshape(1, D5), ln_beta.reshape(1, D5),
      W1, b1.reshape(1, D4), W2, b2.reshape(1, H))


_PCH = 128
_NPCH = K // _PCH
_NVL = L // _LANES
_PAIRS_PER_W = NPAIR // _NW


def _sc_pack_body(gid_hbm, msk_hbm, event_hbm, emp_hbm,
                  states_hbm, mout_hbm,
                  ids_v, msk_v, idxl_v, chunk_v, emp_v, mko_v, sem):
    wid = lax.axis_index("s") * _NC + lax.axis_index("c")
    iota = lax.iota(jnp.int32, _LANES)
    zero16 = jnp.zeros((_LANES,), jnp.int32)

    for p in range(_PAIRS_PER_W):
        pair = wid + p * _NW
        b = pair // S
        s = pair % S
        pltpu.sync_copy(gid_hbm.at[b], ids_v)
        pltpu.sync_copy(msk_hbm.at[b], msk_v)
        pltpu.sync_copy(emp_hbm.at[s], emp_v)

        sval = s + 1

        def count_body(j, c):
            ids16 = ids_v[pl.ds(j * _LANES, _LANES)]
            mk16 = msk_v[pl.ds(j * _LANES, _LANES)]
            m = (ids16 == sval) & (mk16 != 0)
            return c + jnp.sum(jnp.where(m, 1, 0))

        count = lax.fori_loop(0, _NVL, count_body, jnp.int32(0))
        start = jnp.maximum(count - K, 0)
        n = count - start

        def zidx_body(j, _):
            idxl_v[pl.ds(j * _LANES, _LANES)] = j * _LANES + iota
            return 0

        lax.fori_loop(0, K // _LANES, zidx_body, 0)

        def rank_body(j, c):
            ids16 = ids_v[pl.ds(j * _LANES, _LANES)]
            mk16 = msk_v[pl.ds(j * _LANES, _LANES)]
            m = (ids16 == sval) & (mk16 != 0)
            mi = jnp.where(m, 1, 0)
            rank = plsc.cumsum(mi) + c - 1
            slot = rank - start
            wm = m & (slot >= 0)
            slot_c = jnp.maximum(slot, 0)
            gidx = b * L + j * _LANES + iota
            plsc.store_scatter(idxl_v, [slot_c], gidx, mask=wm)
            return c + jnp.sum(mi)

        lax.fori_loop(0, _NVL, rank_body, jnp.int32(0))

        for c4 in range(_NPCH):
            k0 = c4 * _PCH

            @pl.when(k0 < n)
            def _(c4=c4, k0=k0):
                pltpu.async_copy(event_hbm.at[idxl_v.at[pl.ds(k0, _PCH)]],
                                 chunk_v.at[c4], sem)

        for c4 in range(_NPCH):
            k0 = c4 * _PCH

            @pl.when(k0 < n)
            def _(c4=c4, k0=k0):
                pltpu.make_async_copy(
                    event_hbm.at[idxl_v.at[pl.ds(k0, _PCH)]],
                    chunk_v.at[c4], sem).wait()

            if c4 == 0:
                @pl.when(n == 0)
                def _():
                    for j2 in range(H // _LANES):
                        sl = pl.ds(j2 * _LANES, _LANES)
                        chunk_v[0, 0, sl] = emp_v[sl]

            if c4 == 0:
                pltpu.sync_copy(chunk_v.at[0],
                                states_hbm.at[pl.ds(pair * K, _PCH)])
            else:
                @pl.when(k0 < n)
                def _(c4=c4, k0=k0):
                    pltpu.sync_copy(chunk_v.at[c4],
                                    states_hbm.at[pl.ds(pair * K + k0, _PCH)])

        def mk_body(j, _):
            k16 = j * _LANES + iota
            mv = (k16 < n) | ((k16 == 0) & (n == 0))
            mko_v[pl.ds(j * _LANES, _LANES)] = jnp.where(mv, 1, 0)
            return 0

        lax.fori_loop(0, K // _LANES, mk_body, 0)
        pltpu.sync_copy(mko_v, mout_hbm.at[pair])


def _sc_pack(gid, maskI, event, empty_tokens):
    mesh = plsc.VectorSubcoreMesh(core_axis_name="c", subcore_axis_name="s")
    fn = functools.partial(
        pl.kernel, mesh=mesh,
        out_type=[jax.ShapeDtypeStruct((NPAIR * K, H), jnp.float32),
                  jax.ShapeDtypeStruct((NPAIR, K), jnp.int32)],
        compiler_params=pltpu.CompilerParams(needs_layout_passes=False),
        scratch_types=[
            pltpu.VMEM((L,), jnp.int32),
            pltpu.VMEM((L,), jnp.int32),
            pltpu.VMEM((K,), jnp.int32),
            pltpu.VMEM((_NPCH, _PCH, H), jnp.float32),
            pltpu.VMEM((H,), jnp.float32),
            pltpu.VMEM((K,), jnp.int32),
            pltpu.SemaphoreType.DMA,
        ],
    )(_sc_pack_body)
    return fn(gid, maskI, event, empty_tokens)


def _tc_final_body(raw, maskf, pos, sid, out):
    x = raw[0]
    m = maskf[0]
    val = x + pos[...][None, :, :] + sid[...][:, None, :]
    out[0] = jnp.where(m > 0.0, val, 0.0)


def _tc_final(raw, maskf, pos_table, sid_rows):
    return pl.pallas_call(
        _tc_final_body,
        grid=(B,),
        in_specs=[
            pl.BlockSpec((1, S, K, H), lambda i: (i, 0, 0, 0)),
            pl.BlockSpec((1, S, K, 1), lambda i: (i, 0, 0, 0)),
            pl.BlockSpec((K, H), lambda i: (0, 0)),
            pl.BlockSpec((S, H), lambda i: (0, 0)),
        ],
        out_specs=pl.BlockSpec((1, S, K, H), lambda i: (i, 0, 0, 0)),
        out_shape=jax.ShapeDtypeStruct((B, S, K, H), jnp.float32),
    )(raw, maskf, pos_table, sid_rows)


def kernel(history_tokens, history_post_tokens, history_author_tokens,
           history_action_tokens, history_time_gap, history_group_ids,
           history_mask, embed_table, time_gap_table, seq_id_table, pos_table,
           ln_gamma, ln_beta, W1, b1, W2, b2, empty_tokens):
    hist = history_tokens.reshape(NT).astype(jnp.int32)
    post = history_post_tokens.reshape(NT).astype(jnp.int32)
    auth = history_author_tokens.reshape(NT).astype(jnp.int32)
    act = history_action_tokens.reshape(NT).astype(jnp.int32)
    gap = history_time_gap.reshape(NT).astype(jnp.int32)
    gid = history_group_ids.astype(jnp.int32)
    maskI = history_mask.astype(jnp.int32)

    tgpad = jnp.zeros((H, H), jnp.float32).at[:TG + 1].set(time_gap_table)
    xs = _make_sc_gather(NT)(embed_table, hist, post, auth, act)
    event = _tc_mlp(xs, gap.reshape(NT, 1), tgpad,
                    ln_gamma, ln_beta, W1, b1, W2, b2)
    sid_rows = seq_id_table[1:S + 1]
    raw, mout = _sc_pack(gid, maskI, event, empty_tokens)
    maskf = mout.astype(jnp.float32).reshape(B, S, K, 1)
    states = _tc_final(raw.reshape(B, S, K, H), maskf, pos_table, sid_rows)
    seq_mask = (mout != 0).reshape(B, S, K)
    return states, seq_mask


# speedup vs baseline: 7.9652x; 1.0140x over previous
import functools

import jax
import jax.numpy as jnp
from jax import lax
from jax.experimental import pallas as pl
from jax.experimental.pallas import tpu as pltpu
from jax.experimental.pallas import tpu_sc as plsc

B, L, K, H, S, V, TG = 8, 2048, 512, 128, 8, 100000, 64
NT = B * L
NPAIR = B * S
_LANES = 16

_NC = 2
_NS = 16
_NW = _NC * _NS


_GCH = 128


def _make_sc_gather(nt):
    tok_per_w = nt // _NW
    ngch = tok_per_w // _GCH

    def body(embed_hbm, hist_hbm, post_hbm, auth_hbm, act_hbm,
             x0, x1, x2, x3, idx_v, rows_v, sem):
        wid = lax.axis_index("s") * _NC + lax.axis_index("c")
        base = wid * tok_per_w
        srcs = ((hist_hbm, x0), (post_hbm, x1), (auth_hbm, x2), (act_hbm, x3))
        for idx_hbm, out_hbm in srcs:
            for c in range(ngch):
                off = base + c * _GCH
                pltpu.sync_copy(idx_hbm.at[pl.ds(off, _GCH)], idx_v)
                pltpu.async_copy(embed_hbm.at[idx_v], rows_v, sem).wait()
                pltpu.sync_copy(rows_v, out_hbm.at[pl.ds(off, _GCH)])

    mesh = plsc.VectorSubcoreMesh(core_axis_name="c", subcore_axis_name="s")
    xt = jax.ShapeDtypeStruct((nt, H), jnp.float32)
    return functools.partial(
        pl.kernel, mesh=mesh,
        out_type=[xt, xt, xt, xt],
        compiler_params=pltpu.CompilerParams(needs_layout_passes=False),
        scratch_types=[
            pltpu.VMEM((_GCH,), jnp.int32),
            pltpu.VMEM((_GCH, H), jnp.float32),
            pltpu.SemaphoreType.DMA,
        ],
    )(body)


_BT = 2048


def _tc_mlp_body(x0, x1, x2, x3, gap, tgp, gamma, beta, w1, b1, w2, b2, out):
    oh = jnp.where(gap[...] == lax.broadcasted_iota(jnp.int32, (_BT, H), 1),
                   1.0, 0.0)
    x4 = jnp.dot(oh, tgp[...], preferred_element_type=jnp.float32,
                 precision=lax.Precision.HIGHEST)
    x = jnp.concatenate(
        [x0[...], x1[...], x2[...], x3[...], x4], axis=1)
    mu = jnp.mean(x, axis=-1, keepdims=True)
    var = jnp.mean((x - mu) ** 2, axis=-1, keepdims=True)
    xn = (x - mu) * lax.rsqrt(var + 1e-5) * gamma[...] + beta[...]
    h1 = jnp.dot(xn, w1[...], preferred_element_type=jnp.float32) + b1[...]
    h1 = h1 * jax.nn.sigmoid(h1)
    ev = jnp.dot(h1, w2[...], preferred_element_type=jnp.float32) + b2[...]
    out[...] = ev


def _tc_mlp(xs, gap2d, tgpad, ln_gamma, ln_beta, W1, b1, W2, b2, nt=NT):
    D5 = 5 * H
    D4 = 4 * H
    grid = (nt // _BT,)
    xspec = pl.BlockSpec((_BT, H), lambda i: (i, 0))

    def full(shape):
        return pl.BlockSpec(shape, lambda i: tuple(0 for _ in shape))

    return pl.pallas_call(
        _tc_mlp_body,
        grid=grid,
        in_specs=[xspec] * 4 + [pl.BlockSpec((_BT, 1), lambda i: (i, 0)),
                                full((H, H)), full((1, D5)), full((1, D5)),
                                full((D5, D4)), full((1, D4)),
                                full((D4, H)), full((1, H))],
        out_specs=pl.BlockSpec((_BT, H), lambda i: (i, 0)),
        out_shape=jax.ShapeDtypeStruct((nt, H), jnp.float32),
    )(*xs, gap2d, tgpad, ln_gamma.reshape(1, D5), ln_beta.reshape(1, D5),
      W1, b1.reshape(1, D4), W2, b2.reshape(1, H))


_PCH = 128
_NPCH = K // _PCH
_NVL = L // _LANES
_PAIRS_PER_W = NPAIR // _NW


def _sc_pack_body(gid_hbm, msk_hbm, event_hbm, emp_hbm,
                  states_hbm, mout_hbm,
                  ids_v, msk_v, idxl_v, chunk_v, emp_v, mko_v, sem):
    wid = lax.axis_index("s") * _NC + lax.axis_index("c")
    iota = lax.iota(jnp.int32, _LANES)
    zero16 = jnp.zeros((_LANES,), jnp.int32)

    for p in range(_PAIRS_PER_W):
        pair = wid + p * _NW
        b = pair // S
        s = pair % S
        pltpu.sync_copy(gid_hbm.at[b], ids_v)
        pltpu.sync_copy(msk_hbm.at[b], msk_v)
        pltpu.sync_copy(emp_hbm.at[s], emp_v)

        sval = s + 1

        def count_body(j, c):
            ids16 = ids_v[pl.ds(j * _LANES, _LANES)]
            mk16 = msk_v[pl.ds(j * _LANES, _LANES)]
            m = (ids16 == sval) & (mk16 != 0)
            return c + jnp.sum(jnp.where(m, 1, 0))

        count = lax.fori_loop(0, _NVL, count_body, jnp.int32(0))
        start = jnp.maximum(count - K, 0)
        n = count - start

        def zidx_body(j, _):
            idxl_v[pl.ds(j * _LANES, _LANES)] = j * _LANES + iota
            return 0

        lax.fori_loop(0, K // _LANES, zidx_body, 0)

        def rank_body(j, c):
            ids16 = ids_v[pl.ds(j * _LANES, _LANES)]
            mk16 = msk_v[pl.ds(j * _LANES, _LANES)]
            m = (ids16 == sval) & (mk16 != 0)
            mi = jnp.where(m, 1, 0)
            rank = plsc.cumsum(mi) + c - 1
            slot = rank - start
            wm = m & (slot >= 0)
            slot_c = jnp.maximum(slot, 0)
            gidx = b * L + j * _LANES + iota
            plsc.store_scatter(idxl_v, [slot_c], gidx, mask=wm)
            return c + jnp.sum(mi)

        lax.fori_loop(0, _NVL, rank_body, jnp.int32(0))

        for c4 in range(_NPCH):
            k0 = c4 * _PCH

            @pl.when(k0 < n)
            def _(c4=c4, k0=k0):
                pltpu.async_copy(event_hbm.at[idxl_v.at[pl.ds(k0, _PCH)]],
                                 chunk_v.at[c4], sem)

        for c4 in range(_NPCH):
            k0 = c4 * _PCH

            @pl.when(k0 < n)
            def _(c4=c4, k0=k0):
                pltpu.make_async_copy(
                    event_hbm.at[idxl_v.at[pl.ds(k0, _PCH)]],
                    chunk_v.at[c4], sem).wait()

            if c4 == 0:
                @pl.when(n == 0)
                def _():
                    for j2 in range(H // _LANES):
                        sl = pl.ds(j2 * _LANES, _LANES)
                        chunk_v[0, 0, sl] = emp_v[sl]

            if c4 == 0:
                pltpu.sync_copy(chunk_v.at[0],
                                states_hbm.at[pl.ds(pair * K, _PCH)])
            else:
                @pl.when(k0 < n)
                def _(c4=c4, k0=k0):
                    pltpu.sync_copy(chunk_v.at[c4],
                                    states_hbm.at[pl.ds(pair * K + k0, _PCH)])

        def mk_body(j, _):
            k16 = j * _LANES + iota
            mv = (k16 < n) | ((k16 == 0) & (n == 0))
            mko_v[pl.ds(j * _LANES, _LANES)] = jnp.where(mv, 1, 0)
            return 0

        lax.fori_loop(0, K // _LANES, mk_body, 0)
        pltpu.sync_copy(mko_v, mout_hbm.at[pair])


def _sc_pack(gid, maskI, event, empty_tokens):
    mesh = plsc.VectorSubcoreMesh(core_axis_name="c", subcore_axis_name="s")
    fn = functools.partial(
        pl.kernel, mesh=mesh,
        out_type=[jax.ShapeDtypeStruct((NPAIR * K, H), jnp.float32),
                  jax.ShapeDtypeStruct((NPAIR, K), jnp.int32)],
        compiler_params=pltpu.CompilerParams(needs_layout_passes=False),
        scratch_types=[
            pltpu.VMEM((L,), jnp.int32),
            pltpu.VMEM((L,), jnp.int32),
            pltpu.VMEM((K,), jnp.int32),
            pltpu.VMEM((_NPCH, _PCH, H), jnp.float32),
            pltpu.VMEM((H,), jnp.float32),
            pltpu.VMEM((K,), jnp.int32),
            pltpu.SemaphoreType.DMA,
        ],
    )(_sc_pack_body)
    return fn(gid, maskI, event, empty_tokens)


def _tc_final_body(raw, maskf, pos, sid, out):
    x = raw[0]
    m = maskf[0]
    val = x + pos[...][None, :, :] + sid[...][:, None, :]
    out[0] = jnp.where(m > 0.0, val, 0.0)


def _tc_final(raw, maskf, pos_table, sid_rows):
    return pl.pallas_call(
        _tc_final_body,
        grid=(B,),
        in_specs=[
            pl.BlockSpec((1, S, K, H), lambda i: (i, 0, 0, 0)),
            pl.BlockSpec((1, S, K, 1), lambda i: (i, 0, 0, 0)),
            pl.BlockSpec((K, H), lambda i: (0, 0)),
            pl.BlockSpec((S, H), lambda i: (0, 0)),
        ],
        out_specs=pl.BlockSpec((1, S, K, H), lambda i: (i, 0, 0, 0)),
        out_shape=jax.ShapeDtypeStruct((B, S, K, H), jnp.float32),
    )(raw, maskf, pos_table, sid_rows)


def kernel(history_tokens, history_post_tokens, history_author_tokens,
           history_action_tokens, history_time_gap, history_group_ids,
           history_mask, embed_table, time_gap_table, seq_id_table, pos_table,
           ln_gamma, ln_beta, W1, b1, W2, b2, empty_tokens):
    hist = history_tokens.reshape(NT).astype(jnp.int32)
    post = history_post_tokens.reshape(NT).astype(jnp.int32)
    auth = history_author_tokens.reshape(NT).astype(jnp.int32)
    act = history_action_tokens.reshape(NT).astype(jnp.int32)
    gap = history_time_gap.reshape(NT).astype(jnp.int32)
    gid = history_group_ids.astype(jnp.int32)
    maskI = history_mask.astype(jnp.int32)

    tgpad = jnp.zeros((H, H), jnp.float32).at[:TG + 1].set(time_gap_table)
    xs = _make_sc_gather(NT)(embed_table, hist, post, auth, act)
    event = _tc_mlp(xs, gap.reshape(NT, 1), tgpad,
                    ln_gamma, ln_beta, W1, b1, W2, b2)
    sid_rows = seq_id_table[1:S + 1]
    raw, mout = _sc_pack(gid, maskI, event, empty_tokens)
    maskf = mout.astype(jnp.float32).reshape(B, S, K, 1)
    states = _tc_final(raw.reshape(B, S, K, H), maskf, pos_table, sid_rows)
    seq_mask = (mout != 0).reshape(B, S, K)
    return states, seq_mask
